# baseline (speedup 1.0000x reference)
import functools

import jax
import jax.numpy as jnp
from jax import lax
from jax.experimental import pallas as pl
from jax.experimental.pallas import tpu as pltpu
from jax.experimental.pallas import tpu_sc as plsc

VOCAB = 100000
EMB = 300
NOUT = 2
BATCH = 4096
HIST = 50
LANES = 16
NC, NS = 2, 16
NW = NC * NS
NSHARD = NW // NOUT
B_PER_W = BATCH // NSHARD
NGRP = B_PER_W // LANES
MM_BLK = 10000


NBLK = VOCAB // MM_BLK


def _matmul_body(w_ref, wt_ref, o_ref):
    o_ref[0] = jnp.dot(w_ref[...], wt_ref[...],
                       preferred_element_type=jnp.float32).T


def _project(weight, wt16):
    return pl.pallas_call(
        _matmul_body,
        grid=(NBLK,),
        in_specs=[
            pl.BlockSpec((MM_BLK, EMB), lambda i: (i, 0)),
            pl.BlockSpec((EMB, LANES), lambda i: (0, 0)),
        ],
        out_specs=pl.BlockSpec((1, LANES, MM_BLK), lambda i: (i, 0, 0)),
        out_shape=jax.ShapeDtypeStruct((NBLK, LANES, MM_BLK), jnp.float32),
        compiler_params=pltpu.CompilerParams(vmem_limit_bytes=112 * 2**20),
    )(weight, wt16)


def _pool_body(pt_hbm, xt_hbm, out_hbm, tbl_v, xt_v, out_v, scale_v):
    wid = lax.axis_index("s") * NC + lax.axis_index("c")
    col = wid % NOUT
    r0 = (wid // NOUT) * B_PER_W
    for k in range(NBLK):
        pltpu.sync_copy(pt_hbm.at[k, col], tbl_v.at[pl.ds(k * MM_BLK, MM_BLK)])
    pltpu.sync_copy(xt_hbm.at[:, pl.ds(r0, B_PER_W)], xt_v)
    scale_v[...] = jnp.full((LANES,), 1.0 / HIST, jnp.float32)

    @pl.loop(0, NGRP)
    def _grp(g):
        idx0 = xt_v[0, pl.ds(g * LANES, LANES)]
        acc = plsc.load_gather(tbl_v, [idx0])
        for l in range(1, HIST):
            idx = xt_v[l, pl.ds(g * LANES, LANES)]
            acc = acc + plsc.load_gather(tbl_v, [idx])
        out_v[pl.ds(g * LANES, LANES)] = acc * scale_v[...]

    pltpu.sync_copy(out_v, out_hbm.at[col, pl.ds(r0, B_PER_W)])


@functools.cache
def _pool():
    return pl.kernel(
        _pool_body,
        out_type=jax.ShapeDtypeStruct((NOUT, BATCH), jnp.float32),
        mesh=plsc.VectorSubcoreMesh(core_axis_name="c", subcore_axis_name="s",
                                    num_cores=NC, num_subcores=NS),
        compiler_params=pltpu.CompilerParams(use_tc_tiling_on_sc=False,
                                             needs_layout_passes=False),
        scratch_types=[
            pltpu.VMEM((VOCAB,), jnp.float32),
            pltpu.VMEM((HIST, B_PER_W), jnp.int32),
            pltpu.VMEM((B_PER_W,), jnp.float32),
            pltpu.VMEM((LANES,), jnp.float32),
        ],
    )


def kernel(x, weight, W_out):
    wt16 = jnp.zeros((EMB, LANES), jnp.float32).at[:, :NOUT].set(W_out.T)
    pt = _project(weight, wt16)
    xt = x.astype(jnp.int32).T
    pooled = _pool()(pt, xt)
    return pooled.T


# speedup vs baseline: 2.5855x; 2.5855x over previous
import functools

import jax
import jax.numpy as jnp
from jax import lax
from jax.experimental import pallas as pl
from jax.experimental.pallas import tpu as pltpu
from jax.experimental.pallas import tpu_sc as plsc

VOCAB = 100000
EMB = 300
NOUT = 2
BATCH = 4096
HIST = 50
LANES = 16
NC, NS = 2, 16
NW = NC * NS
NSHARD = NW // NOUT
B_PER_W = BATCH // NSHARD
NGRP = B_PER_W // LANES
K_BLK = 48
K_STEPS = -(-EMB // K_BLK)
K_PAD = K_BLK * K_STEPS


def _matmul_body(wt_ref, w_ref, o_ref):
    part = lax.dot_general(wt_ref[...], w_ref[...],
                           (((0,), (0,)), ((), ())),
                           preferred_element_type=jnp.float32)

    @pl.when(pl.program_id(0) == 0)
    def _():
        o_ref[...] = part

    @pl.when(pl.program_id(0) > 0)
    def _():
        o_ref[...] = o_ref[...] + part


def _project(wT, wtp):
    return pl.pallas_call(
        _matmul_body,
        grid=(K_STEPS,),
        in_specs=[
            pl.BlockSpec((K_BLK, LANES), lambda i: (i, 0)),
            pl.BlockSpec((K_BLK, VOCAB), lambda i: (i, 0)),
        ],
        out_specs=pl.BlockSpec((LANES, VOCAB), lambda i: (0, 0)),
        out_shape=jax.ShapeDtypeStruct((LANES, VOCAB), jnp.float32),
        compiler_params=pltpu.CompilerParams(vmem_limit_bytes=56 * 2**20),
    )(wtp, wT)


def _pool_body(pt_hbm, xt_hbm, out_hbm, tbl_v, xt_v, out_v, scale_v):
    wid = lax.axis_index("s") * NC + lax.axis_index("c")
    col = wid % NOUT
    r0 = (wid // NOUT) * B_PER_W
    pltpu.sync_copy(pt_hbm.at[col], tbl_v)
    pltpu.sync_copy(xt_hbm.at[:, pl.ds(r0, B_PER_W)], xt_v)
    scale_v[...] = jnp.full((LANES,), 1.0 / HIST, jnp.float32)

    @pl.loop(0, NGRP)
    def _grp(g):
        idx0 = xt_v[0, pl.ds(g * LANES, LANES)]
        acc = plsc.load_gather(tbl_v, [idx0])
        for l in range(1, HIST):
            idx = xt_v[l, pl.ds(g * LANES, LANES)]
            acc = acc + plsc.load_gather(tbl_v, [idx])
        out_v[pl.ds(g * LANES, LANES)] = acc * scale_v[...]

    pltpu.sync_copy(out_v, out_hbm.at[col, pl.ds(r0, B_PER_W)])


@functools.cache
def _pool():
    return pl.kernel(
        _pool_body,
        out_type=jax.ShapeDtypeStruct((NOUT, BATCH), jnp.float32),
        mesh=plsc.VectorSubcoreMesh(core_axis_name="c", subcore_axis_name="s",
                                    num_cores=NC, num_subcores=NS),
        compiler_params=pltpu.CompilerParams(use_tc_tiling_on_sc=False,
                                             needs_layout_passes=False),
        scratch_types=[
            pltpu.VMEM((VOCAB,), jnp.float32),
            pltpu.VMEM((HIST, B_PER_W), jnp.int32),
            pltpu.VMEM((B_PER_W,), jnp.float32),
            pltpu.VMEM((LANES,), jnp.float32),
        ],
    )


def kernel(x, weight, W_out):
    wtp = jnp.zeros((K_PAD, LANES), jnp.float32).at[:EMB, :NOUT].set(W_out.T)
    pt = _project(weight.T, wtp)
    xt = x.astype(jnp.int32).T
    pooled = _pool()(pt, xt)
    return pooled.T


# speedup vs baseline: 2.9210x; 1.1298x over previous
import functools

import jax
import jax.numpy as jnp
from jax import lax
from jax.experimental import pallas as pl
from jax.experimental.pallas import tpu as pltpu
from jax.experimental.pallas import tpu_sc as plsc

VOCAB = 100000
EMB = 300
NOUT = 2
BATCH = 4096
HIST = 50
LANES = 16
NC, NS = 2, 16
NW = NC * NS
NSHARD = NW // NOUT
B_PER_W = BATCH // NSHARD
NGRP = B_PER_W // LANES
K_BLK = 64
K_STEPS = -(-EMB // K_BLK)
K_PAD = K_BLK * K_STEPS


def _matmul_body(wt_ref, w_ref, o0_ref, o1_ref):
    part = lax.dot_general(wt_ref[...], w_ref[...],
                           (((0,), (0,)), ((), ())),
                           preferred_element_type=jnp.float32)

    @pl.when(pl.program_id(0) == 0)
    def _():
        o0_ref[...] = part[0]
        o1_ref[...] = part[1]

    @pl.when(pl.program_id(0) > 0)
    def _():
        o0_ref[...] = o0_ref[...] + part[0]
        o1_ref[...] = o1_ref[...] + part[1]


def _project(wT, wtp):
    return pl.pallas_call(
        _matmul_body,
        grid=(K_STEPS,),
        in_specs=[
            pl.BlockSpec((K_BLK, 8), lambda i: (i, 0)),
            pl.BlockSpec((K_BLK, VOCAB), lambda i: (i, 0)),
        ],
        out_specs=[pl.BlockSpec((VOCAB,), lambda i: (0,)),
                   pl.BlockSpec((VOCAB,), lambda i: (0,))],
        out_shape=[jax.ShapeDtypeStruct((VOCAB,), jnp.float32),
                   jax.ShapeDtypeStruct((VOCAB,), jnp.float32)],
        compiler_params=pltpu.CompilerParams(vmem_limit_bytes=56 * 2**20),
    )(wtp, wT)


def _pool_body(p0_hbm, p1_hbm, xt_hbm, out_hbm, tbl_v, xt_v, out_v, scale_v):
    wid = lax.axis_index("s") * NC + lax.axis_index("c")
    col = wid % NOUT
    r0 = (wid // NOUT) * B_PER_W

    @pl.when(col == 0)
    def _():
        pltpu.sync_copy(p0_hbm, tbl_v)

    @pl.when(col == 1)
    def _():
        pltpu.sync_copy(p1_hbm, tbl_v)
    pltpu.sync_copy(xt_hbm.at[:, pl.ds(r0, B_PER_W)], xt_v)
    scale_v[...] = jnp.full((LANES,), 1.0 / HIST, jnp.float32)

    @pl.loop(0, NGRP)
    def _grp(g):
        idx0 = xt_v[0, pl.ds(g * LANES, LANES)]
        acc = plsc.load_gather(tbl_v, [idx0])
        for l in range(1, HIST):
            idx = xt_v[l, pl.ds(g * LANES, LANES)]
            acc = acc + plsc.load_gather(tbl_v, [idx])
        out_v[pl.ds(g * LANES, LANES)] = acc * scale_v[...]

    pltpu.sync_copy(out_v, out_hbm.at[col, pl.ds(r0, B_PER_W)])


@functools.cache
def _pool():
    return pl.kernel(
        _pool_body,
        out_type=jax.ShapeDtypeStruct((NOUT, BATCH), jnp.float32),
        mesh=plsc.VectorSubcoreMesh(core_axis_name="c", subcore_axis_name="s",
                                    num_cores=NC, num_subcores=NS),
        compiler_params=pltpu.CompilerParams(use_tc_tiling_on_sc=False,
                                             needs_layout_passes=False),
        scratch_types=[
            pltpu.VMEM((VOCAB,), jnp.float32),
            pltpu.VMEM((HIST, B_PER_W), jnp.int32),
            pltpu.VMEM((B_PER_W,), jnp.float32),
            pltpu.VMEM((LANES,), jnp.float32),
        ],
    )


def kernel(x, weight, W_out):
    wtp = jnp.zeros((K_PAD, 8), jnp.float32).at[:EMB, :NOUT].set(W_out.T)
    p0, p1 = _project(weight.T, wtp)
    xt = x.astype(jnp.int32).T
    pooled = _pool()(p0, p1, xt)
    return pooled.T


# speedup vs baseline: 2.9614x; 1.0138x over previous
import functools

import jax
import jax.numpy as jnp
from jax import lax
from jax.experimental import pallas as pl
from jax.experimental.pallas import tpu as pltpu
from jax.experimental.pallas import tpu_sc as plsc

VOCAB = 100000
EMB = 300
NOUT = 2
BATCH = 4096
HIST = 50
LANES = 16
NC, NS = 2, 16
NW = NC * NS
NSHARD = NW // NOUT
B_PER_W = BATCH // NSHARD
NGRP = B_PER_W // LANES
K_BLK = 64
K_STEPS = -(-EMB // K_BLK)
K_PAD = K_BLK * K_STEPS


def _matmul_body(wt_ref, w_ref, o0_ref, o1_ref):
    part = lax.dot_general(wt_ref[...], w_ref[...],
                           (((0,), (0,)), ((), ())),
                           preferred_element_type=jnp.float32)

    @pl.when(pl.program_id(0) == 0)
    def _():
        o0_ref[...] = part[0]
        o1_ref[...] = part[1]

    @pl.when(pl.program_id(0) > 0)
    def _():
        o0_ref[...] = o0_ref[...] + part[0]
        o1_ref[...] = o1_ref[...] + part[1]


def _project(wT, wtp):
    return pl.pallas_call(
        _matmul_body,
        grid=(K_STEPS,),
        in_specs=[
            pl.BlockSpec((K_BLK, 8), lambda i: (i, 0)),
            pl.BlockSpec((K_BLK, VOCAB), lambda i: (i, 0)),
        ],
        out_specs=[pl.BlockSpec((VOCAB,), lambda i: (0,)),
                   pl.BlockSpec((VOCAB,), lambda i: (0,))],
        out_shape=[jax.ShapeDtypeStruct((VOCAB,), jnp.float32),
                   jax.ShapeDtypeStruct((VOCAB,), jnp.float32)],
        compiler_params=pltpu.CompilerParams(vmem_limit_bytes=56 * 2**20),
    )(wtp, wT)


def _pool_body(p0_hbm, p1_hbm, xt_hbm, out_hbm, tbl_v, xt_v, out_v, scale_v,
               tbl_sem, xt_sem):
    wid = lax.axis_index("s") * NC + lax.axis_index("c")
    col = wid % NOUT
    r0 = (wid // NOUT) * B_PER_W

    xt_copy = pltpu.async_copy(xt_hbm.at[:, pl.ds(r0, B_PER_W)], xt_v, xt_sem)

    @pl.when(col == 0)
    def _():
        pltpu.async_copy(p0_hbm, tbl_v, tbl_sem)

    @pl.when(col == 1)
    def _():
        pltpu.async_copy(p1_hbm, tbl_v, tbl_sem)
    scale_v[...] = jnp.full((LANES,), 1.0 / HIST, jnp.float32)
    xt_copy.wait()
    pltpu.make_async_copy(p0_hbm, tbl_v, tbl_sem).wait()

    @pl.loop(0, NGRP)
    def _grp(g):
        idx0 = xt_v[0, pl.ds(g * LANES, LANES)]
        acc = plsc.load_gather(tbl_v, [idx0])
        for l in range(1, HIST):
            idx = xt_v[l, pl.ds(g * LANES, LANES)]
            acc = acc + plsc.load_gather(tbl_v, [idx])
        out_v[pl.ds(g * LANES, LANES)] = acc * scale_v[...]

    pltpu.sync_copy(out_v, out_hbm.at[col, pl.ds(r0, B_PER_W)])


@functools.cache
def _pool():
    return pl.kernel(
        _pool_body,
        out_type=jax.ShapeDtypeStruct((NOUT, BATCH), jnp.float32),
        mesh=plsc.VectorSubcoreMesh(core_axis_name="c", subcore_axis_name="s",
                                    num_cores=NC, num_subcores=NS),
        compiler_params=pltpu.CompilerParams(use_tc_tiling_on_sc=False,
                                             needs_layout_passes=False),
        scratch_types=[
            pltpu.VMEM((VOCAB,), jnp.float32),
            pltpu.VMEM((HIST, B_PER_W), jnp.int32),
            pltpu.VMEM((B_PER_W,), jnp.float32),
            pltpu.VMEM((LANES,), jnp.float32),
            pltpu.SemaphoreType.DMA,
            pltpu.SemaphoreType.DMA,
        ],
    )


def kernel(x, weight, W_out):
    wtp = jnp.zeros((K_PAD, 8), jnp.float32).at[:EMB, :NOUT].set(W_out.T)
    p0, p1 = _project(weight.T, wtp)
    xt = x.astype(jnp.int32).T
    pooled = _pool()(p0, p1, xt)
    return pooled.T
